# Initial kernel scaffold; baseline (speedup 1.0000x reference)
import jax
import jax.numpy as jnp
from jax.experimental import pallas as pl


def kernel(inputs, embedding_weight):
    raise NotImplementedError("write your pallas kernel here")


# speedup vs baseline: 1.1873x; 1.1873x over previous
import functools

import jax
import jax.numpy as jnp
from jax import lax
from jax.experimental import pallas as pl
from jax.experimental.pallas import tpu as pltpu
from jax.experimental.pallas import tpu_sc as plsc

_N = 8192
_D = 256
_B = 8192
_COMMIT = 0.25

_R1 = 1024
_C1 = 2048


def _k1_body(x_ref, e_ref, idx_ref, bv_ref, bi_ref):
    c = pl.program_id(0)
    r = pl.program_id(1)
    x = x_ref[...]
    e = e_ref[...]
    xb = (x + x).astype(jnp.bfloat16).astype(jnp.float32)
    mm = lax.dot_general(xb, e, (((1,), (1,)), ((), ())),
                         preferred_element_type=jnp.float32)
    x2 = jnp.sum(x * x, axis=1, keepdims=True)
    e2 = jnp.sum(e * e, axis=1)[None, :]
    d = (x2 + e2) - mm
    m = jnp.min(d, axis=1, keepdims=True)
    iota = (lax.broadcasted_iota(jnp.int32, (_R1, _C1), 1)
            + c * _C1).astype(jnp.float32)
    ii = jnp.min(jnp.where(d == m, iota, jnp.float32(2 ** 30)),
                 axis=1, keepdims=True)
    rows = pl.ds(r * _R1, _R1)

    @pl.when(c == 0)
    def _():
        bv_ref[rows, :] = m
        bi_ref[rows, :] = ii

    @pl.when(c > 0)
    def _():
        bv = bv_ref[rows, :]
        bi = bi_ref[rows, :]
        upd = m < bv
        bv_ref[rows, :] = jnp.where(upd, m, bv)
        bi_ref[rows, :] = jnp.where(upd, ii, bi)

    idx_ref[...] = bi_ref[rows, :].astype(jnp.int32)


def _argmin_indices(x, e):
    return pl.pallas_call(
        _k1_body,
        grid=(_N // _C1, _B // _R1),
        in_specs=[
            pl.BlockSpec((_R1, _D), lambda c, r: (r, 0)),
            pl.BlockSpec((_C1, _D), lambda c, r: (c, 0)),
        ],
        out_specs=pl.BlockSpec((_R1, 1), lambda c, r: (r, 0)),
        out_shape=jax.ShapeDtypeStruct((_B, 1), jnp.int32),
        scratch_shapes=[
            pltpu.VMEM((_B, 1), jnp.float32),
            pltpu.VMEM((_B, 1), jnp.float32),
        ],
    )(x, e)


_R2 = 256


def _k2_body(idx_ref, enc_ref, pp_ref, counts_ref):
    r = pl.program_id(0)
    idx = idx_ref[...]
    iota = lax.broadcasted_iota(jnp.int32, (_R2, _N), 1)
    oh = (iota == idx).astype(jnp.float32)
    enc_ref[...] = oh
    cs = jnp.sum(oh, axis=0, keepdims=True)

    @pl.when(r == 0)
    def _():
        counts_ref[...] = cs

    @pl.when(r > 0)
    def _():
        counts_ref[...] = counts_ref[...] + cs

    @pl.when(r == pl.num_programs(0) - 1)
    def _():
        p = counts_ref[...] * jnp.float32(1.0 / _B)
        ent = jnp.sum(p * jnp.log(p + 1e-10), keepdims=True)
        pp_ref[...] = jnp.exp(-ent)


def _encodings_perplexity(idx2):
    enc, pp, _counts = pl.pallas_call(
        _k2_body,
        grid=(_B // _R2,),
        in_specs=[pl.BlockSpec((_R2, 1), lambda r: (r, 0))],
        out_specs=[
            pl.BlockSpec((_R2, _N), lambda r: (r, 0)),
            pl.BlockSpec((1, 1), lambda r: (0, 0)),
            pl.BlockSpec((1, _N), lambda r: (0, 0)),
        ],
        out_shape=[
            jax.ShapeDtypeStruct((_B, _N), jnp.float32),
            jax.ShapeDtypeStruct((1, 1), jnp.float32),
            jax.ShapeDtypeStruct((1, _N), jnp.float32),
        ],
    )(idx2)
    return enc, pp


_NC = 2
_NS = 16
_NW = _NC * _NS
_BPW = _B // _NW


def _gather_rows(table, idx):
    mesh = plsc.VectorSubcoreMesh(core_axis_name="c", subcore_axis_name="s")

    @functools.partial(
        pl.kernel, mesh=mesh,
        out_type=jax.ShapeDtypeStruct((_B, _D), jnp.float32),
        scratch_types=[
            pltpu.VMEM((_BPW,), jnp.int32),
            pltpu.VMEM((_BPW, _D), jnp.float32),
            pltpu.SemaphoreType.DMA,
        ],
    )
    def k(table_hbm, idx_hbm, out_hbm, idx_v, rows_v, sem):
        wid = lax.axis_index("s") * _NC + lax.axis_index("c")
        base = wid * _BPW
        pltpu.sync_copy(idx_hbm.at[pl.ds(base, _BPW)], idx_v)
        pltpu.async_copy(table_hbm.at[idx_v], rows_v, sem).wait()
        pltpu.sync_copy(rows_v, out_hbm.at[pl.ds(base, _BPW)])

    return k(table, idx)


_R3 = 1024


def _k3_body(x_ref, q_ref, out_ref, ls_ref):
    i = pl.program_id(0)
    x = x_ref[...]
    q = q_ref[...]
    dq = q - x
    out_ref[...] = x + dq
    s = jnp.sum(dq * dq, keepdims=True)

    @pl.when(i == 0)
    def _():
        ls_ref[...] = s

    @pl.when(i > 0)
    def _():
        ls_ref[...] = ls_ref[...] + s


def _st_and_loss(x, q):
    return pl.pallas_call(
        _k3_body,
        grid=(_B // _R3,),
        in_specs=[
            pl.BlockSpec((_R3, _D), lambda i: (i, 0)),
            pl.BlockSpec((_R3, _D), lambda i: (i, 0)),
        ],
        out_specs=[
            pl.BlockSpec((_R3, _D), lambda i: (i, 0)),
            pl.BlockSpec((1, 1), lambda i: (0, 0)),
        ],
        out_shape=[
            jax.ShapeDtypeStruct((_B, _D), jnp.float32),
            jax.ShapeDtypeStruct((1, 1), jnp.float32),
        ],
    )(x, q)


def kernel(inputs, embedding_weight):
    x = jnp.transpose(inputs, (0, 2, 3, 1)).reshape(_B, _D)
    idx2 = _argmin_indices(x, embedding_weight)
    q = _gather_rows(embedding_weight, idx2.reshape(_B))
    enc, pp = _encodings_perplexity(idx2)
    qst, loss_sum = _st_and_loss(x, q)
    loss = (_COMMIT / (_B * _D)) * loss_sum[0, 0]
    quantized_st = jnp.transpose(qst.reshape(8, 32, 32, _D), (0, 3, 1, 2))
    return (loss, quantized_st, pp[0, 0], enc)
